# baseline (speedup 1.0000x reference)
import jax
import jax.numpy as jnp
from jax.experimental import pallas as pl
from jax.experimental.pallas import tpu as pltpu

_F32 = jnp.float32
_BF16 = jnp.bfloat16


def _dot(a, b):
    return jnp.dot(a.astype(_BF16), b.astype(_BF16),
                   preferred_element_type=_F32)


def _dot_t(a, b):
    return jax.lax.dot_general(a.astype(_BF16), b.astype(_BF16),
                               (((0,), (0,)), ((), ())),
                               preferred_element_type=_F32)


def _dot_nt(a, b):
    return jax.lax.dot_general(a.astype(_BF16), b.astype(_BF16),
                               (((1,), (1,)), ((), ())),
                               preferred_element_type=_F32)


def _conv0(ht_ref, xin_ref, w0_ref, b0_ref,
           hbt_ref, de_ref, x_ref, dv_ref, xacc, dvacc):
    i = pl.program_id(0)

    @pl.when(i == 0)
    def _():
        xacc[...] = jnp.zeros_like(xacc)
        dvacc[...] = jnp.zeros_like(dvacc)

    ht = ht_ref[...]
    hbt_ref[...] = ht.astype(_BF16)
    e0 = _dot(ht, xin_ref[...])
    de = jnp.clip(jnp.sum(ht, axis=1, keepdims=True), 1e-6, None)
    de_ref[...] = de
    g0 = _dot_nt(e0 / de, w0_ref[...])
    xacc[...] += _dot_t(ht, g0)
    dvacc[...] += _dot_t(ht, jnp.ones_like(ht[:, :1], _F32))

    @pl.when(i == pl.num_programs(0) - 1)
    def _():
        dv = jnp.clip(dvacc[...], 1e-6, None)
        dv_ref[...] = dv
        x_ref[...] = jnp.maximum(
            xacc[...] / dv + b0_ref[...], 0.0).astype(_BF16)


def _make_mega(bm, bn, m, n, d, k):
    nsteps = m // bm
    assert nsteps == n // bn

    def mega(ht_ref, x_ref, dv_ref, de_ref, wp_ref, bp_ref, w1_ref,
             b1_ref, wr_ref, br_ref, out_ref,
             s_scr, bp_scr, acc_scr, x2_scr, dv2_scr):
        xn2_scr = x2_scr
        p = pl.program_id(0)
        s = pl.program_id(1)
        last = nsteps - 1
        er = pl.ds(s * bm, bm)
        nr = pl.ds(s * bn, bn)

        @pl.when(p == 0)
        def _phase_c():
            @pl.when(s == 0)
            def _():
                acc_scr[...] = jnp.zeros_like(acc_scr)

            ht = ht_ref[...]
            ep = _dot(ht, x_ref[...]) / de_ref[er, :]
            acc_scr[...] += _dot_t(ht, ep)

        @pl.when(p == 1)
        def _phase_d():
            @pl.when(s == 0)
            def _():
                x2_scr[...] = jnp.zeros_like(x2_scr)
                dv2_scr[...] = jnp.zeros_like(dv2_scr)

            dvb = dv_ref[nr, :]
            t = acc_scr[nr, :] / dvb
            logits = _dot_nt(t, wp_ref[...]) + bp_ref[...]
            mx = jnp.max(logits, axis=1, keepdims=True)
            ex = jnp.exp(logits - mx)
            sm = ex / jnp.sum(ex, axis=1, keepdims=True)
            s_scr[nr, :] = sm.astype(_BF16)
            x2_scr[...] += _dot_t(sm, x_ref[nr, :])
            dv2_scr[...] += _dot_t(sm, dvb)

        @pl.when(p == 2)
        def _phase_e():
            @pl.when(s == 0)
            def _():
                x2_scr[...] = _dot_nt(x2_scr[...], w1_ref[...])

            bp_scr[nr, :] = _dot(s_scr[nr, :], x2_scr[...]).astype(_BF16)

        @pl.when(p == 3)
        def _phase_f():
            @pl.when(s == 0)
            def _():
                acc_scr[...] = jnp.zeros_like(acc_scr)

            ht = ht_ref[...]
            a = _dot(ht, bp_scr[...]) / de_ref[er, :]
            acc_scr[...] += _dot_t(ht, a)

        @pl.when(p == 4)
        def _phase_g():
            @pl.when(s == 0)
            def _():
                xn2_scr[...] = jnp.zeros_like(xn2_scr)

            xn2_scr[...] += _dot_t(s_scr[nr, :], acc_scr[nr, :])

            @pl.when(s == last)
            def _():
                dv2 = jnp.clip(dv2_scr[...], 1e-6, None)
                x3 = jnp.maximum(xn2_scr[...] / dv2 + b1_ref[...], 0.0)
                emb = jnp.sum(x3, axis=0, keepdims=True) * (1.0 / k)
                out_ref[...] = _dot_nt(emb, wr_ref[...]) + br_ref[...]

    return mega


def _pick_block(dim, cands):
    for b in cands:
        if dim % b == 0:
            return b
    return dim


def kernel(node_features, incidence, W0, b0, Wp, bp, W1, b1, Wr, br):
    n, m = incidence.shape
    d = node_features.shape[1]
    k = Wp.shape[0]
    o = Wr.shape[0]
    bm = _pick_block(m, (80, 40, 8))
    bm2 = _pick_block(m, (200, 100, 40, 8))
    bn = (n * bm2) // m
    f32, bf16 = jnp.float32, jnp.bfloat16

    ht = incidence.T
    nsteps = m // bm2

    hbt, de, x, dv = pl.pallas_call(
        _conv0,
        grid=(m // bm,),
        in_specs=[pl.BlockSpec((bm, n), lambda i: (i, 0)),
                  pl.BlockSpec((n, d), lambda i: (0, 0)),
                  pl.BlockSpec((d, d), lambda i: (0, 0)),
                  pl.BlockSpec((1, d), lambda i: (0, 0))],
        out_specs=[pl.BlockSpec((bm, n), lambda i: (i, 0)),
                   pl.BlockSpec((bm, 1), lambda i: (i, 0)),
                   pl.BlockSpec((n, d), lambda i: (0, 0)),
                   pl.BlockSpec((n, 1), lambda i: (0, 0))],
        out_shape=[jax.ShapeDtypeStruct((m, n), bf16),
                   jax.ShapeDtypeStruct((m, 1), f32),
                   jax.ShapeDtypeStruct((n, d), bf16),
                   jax.ShapeDtypeStruct((n, 1), f32)],
        scratch_shapes=[pltpu.VMEM((n, d), f32), pltpu.VMEM((n, 1), f32)],
    )(ht, node_features, W0, b0[None, :])

    def ht_index(p, s):
        use = (p == 0) | (p == 3)
        return (jnp.where(use, s, nsteps - 1), 0)

    def const_index(p, s):
        return (0, 0)

    mega = _make_mega(bm2, bn, m, n, d, k)
    out = pl.pallas_call(
        mega,
        grid=(5, nsteps),
        in_specs=[pl.BlockSpec((bm2, n), ht_index),
                  pl.BlockSpec((n, d), const_index),
                  pl.BlockSpec((n, 1), const_index),
                  pl.BlockSpec((m, 1), const_index),
                  pl.BlockSpec((k, d), const_index),
                  pl.BlockSpec((1, k), const_index),
                  pl.BlockSpec((d, d), const_index),
                  pl.BlockSpec((1, d), const_index),
                  pl.BlockSpec((o, d), const_index),
                  pl.BlockSpec((1, o), const_index)],
        out_specs=pl.BlockSpec((1, o), const_index),
        out_shape=jax.ShapeDtypeStruct((1, o), f32),
        scratch_shapes=[pltpu.VMEM((n, k), bf16),
                        pltpu.VMEM((n, d), bf16),
                        pltpu.VMEM((n, d), f32),
                        pltpu.VMEM((k, d), f32),
                        pltpu.VMEM((k, 1), f32)],
    )(hbt, x, dv, de, Wp, bp[None, :], W1,
      b1[None, :], Wr, br[None, :])

    return out[0]


# speedup vs baseline: 1.2451x; 1.2451x over previous
import jax
import jax.numpy as jnp
from jax.experimental import pallas as pl
from jax.experimental.pallas import tpu as pltpu

_F32 = jnp.float32
_BF16 = jnp.bfloat16


def _dot(a, b):
    return jnp.dot(a.astype(_BF16), b.astype(_BF16),
                   preferred_element_type=_F32)


def _dot_t(a, b):
    return jax.lax.dot_general(a.astype(_BF16), b.astype(_BF16),
                               (((0,), (0,)), ((), ())),
                               preferred_element_type=_F32)


def _conv0(ht_ref, xin_ref, w0t_ref, b0_ref,
           hbt_ref, de_ref, x_ref, dv_ref, xacc, dvacc):
    i = pl.program_id(0)

    @pl.when(i == 0)
    def _():
        xacc[...] = jnp.zeros_like(xacc)
        dvacc[...] = jnp.zeros_like(dvacc)

    ht = ht_ref[...]
    hbt_ref[...] = ht.astype(_BF16)
    e0 = _dot(ht, xin_ref[...])
    de = jnp.clip(jnp.sum(ht, axis=1, keepdims=True), 1e-6, None)
    de_ref[...] = de
    g0 = _dot(e0 / de, w0t_ref[...])
    xacc[...] += _dot_t(ht, g0)
    dvacc[...] += _dot_t(ht, jnp.ones_like(ht[:, :1], _F32))

    @pl.when(i == pl.num_programs(0) - 1)
    def _():
        dv = jnp.clip(dvacc[...], 1e-6, None)
        dv_ref[...] = dv
        x_ref[...] = jnp.maximum(
            xacc[...] / dv + b0_ref[...], 0.0).astype(_BF16)


def _make_mega(bm, bn, m, n, d, k):
    nsteps = m // bm
    assert nsteps == n // bn

    def mega(ht_ref, x_ref, dv_ref, de_ref, wpt_ref, bp_ref, w1t_ref,
             b1_ref, wrt_ref, br_ref, out_ref,
             s_scr, bp_scr, acc_scr, x2_scr, dv2_scr):
        xn2_scr = x2_scr
        p = pl.program_id(0)
        s = pl.program_id(1)
        last = nsteps - 1
        er = pl.ds(s * bm, bm)
        nr = pl.ds(s * bn, bn)

        @pl.when(p == 0)
        def _phase_c():
            @pl.when(s == 0)
            def _():
                acc_scr[...] = jnp.zeros_like(acc_scr)

            ht = ht_ref[...]
            ep = _dot(ht, x_ref[...]) / de_ref[er, :]
            acc_scr[...] += _dot_t(ht, ep)

        @pl.when(p == 1)
        def _phase_d():
            @pl.when(s == 0)
            def _():
                x2_scr[...] = jnp.zeros_like(x2_scr)
                dv2_scr[...] = jnp.zeros_like(dv2_scr)

            dvb = dv_ref[nr, :]
            t = acc_scr[nr, :] / dvb
            logits = _dot(t, wpt_ref[...]) + bp_ref[...]
            mx = jnp.max(logits, axis=1, keepdims=True)
            ex = jnp.exp(logits - mx)
            sm = ex / jnp.sum(ex, axis=1, keepdims=True)
            s_scr[nr, :] = sm.astype(_BF16)
            x2_scr[...] += _dot_t(sm, x_ref[nr, :])
            dv2_scr[...] += _dot_t(sm, dvb)

        @pl.when(p == 2)
        def _phase_e():
            @pl.when(s == 0)
            def _():
                x2_scr[...] = _dot(x2_scr[...], w1t_ref[...])

            bp_scr[nr, :] = _dot(s_scr[nr, :], x2_scr[...]).astype(_BF16)

        @pl.when(p == 3)
        def _phase_f():
            @pl.when(s == 0)
            def _():
                acc_scr[...] = jnp.zeros_like(acc_scr)

            ht = ht_ref[...]
            a = _dot(ht, bp_scr[...]) / de_ref[er, :]
            acc_scr[...] += _dot_t(ht, a)

        @pl.when(p == 4)
        def _phase_g():
            @pl.when(s == 0)
            def _():
                xn2_scr[...] = jnp.zeros_like(xn2_scr)

            xn2_scr[...] += _dot_t(s_scr[nr, :], acc_scr[nr, :])

            @pl.when(s == last)
            def _():
                dv2 = jnp.clip(dv2_scr[...], 1e-6, None)
                x3 = jnp.maximum(xn2_scr[...] / dv2 + b1_ref[...], 0.0)
                emb = jnp.sum(x3, axis=0, keepdims=True) * (1.0 / k)
                out_ref[...] = _dot(emb, wrt_ref[...]) + br_ref[...]

    return mega


def _pick_block(dim, cands):
    for b in cands:
        if dim % b == 0:
            return b
    return dim


def kernel(node_features, incidence, W0, b0, Wp, bp, W1, b1, Wr, br):
    n, m = incidence.shape
    d = node_features.shape[1]
    k = Wp.shape[0]
    o = Wr.shape[0]
    bm = _pick_block(m, (200, 100, 40, 8))
    bm2 = _pick_block(m, (200, 100, 40, 8))
    bn = (n * bm2) // m
    f32, bf16 = jnp.float32, jnp.bfloat16

    ht = incidence.T
    nsteps = m // bm2

    hbt, de, x, dv = pl.pallas_call(
        _conv0,
        grid=(m // bm,),
        in_specs=[pl.BlockSpec((bm, n), lambda i: (i, 0)),
                  pl.BlockSpec((n, d), lambda i: (0, 0)),
                  pl.BlockSpec((d, d), lambda i: (0, 0)),
                  pl.BlockSpec((1, d), lambda i: (0, 0))],
        out_specs=[pl.BlockSpec((bm, n), lambda i: (i, 0)),
                   pl.BlockSpec((bm, 1), lambda i: (i, 0)),
                   pl.BlockSpec((n, d), lambda i: (0, 0)),
                   pl.BlockSpec((n, 1), lambda i: (0, 0))],
        out_shape=[jax.ShapeDtypeStruct((m, n), bf16),
                   jax.ShapeDtypeStruct((m, 1), f32),
                   jax.ShapeDtypeStruct((n, d), bf16),
                   jax.ShapeDtypeStruct((n, 1), f32)],
        scratch_shapes=[pltpu.VMEM((n, d), f32), pltpu.VMEM((n, 1), f32)],
    )(ht, node_features, W0.T, b0[None, :])

    def ht_index(p, s):
        use = (p == 0) | (p == 3)
        return (jnp.where(use, s, nsteps - 1), 0)

    def const_index(p, s):
        return (0, 0)

    mega = _make_mega(bm2, bn, m, n, d, k)
    out = pl.pallas_call(
        mega,
        grid=(5, nsteps),
        in_specs=[pl.BlockSpec((bm2, n), ht_index),
                  pl.BlockSpec((n, d), const_index),
                  pl.BlockSpec((n, 1), const_index),
                  pl.BlockSpec((m, 1), const_index),
                  pl.BlockSpec((d, k), const_index),
                  pl.BlockSpec((1, k), const_index),
                  pl.BlockSpec((d, d), const_index),
                  pl.BlockSpec((1, d), const_index),
                  pl.BlockSpec((d, o), const_index),
                  pl.BlockSpec((1, o), const_index)],
        out_specs=pl.BlockSpec((1, o), const_index),
        out_shape=jax.ShapeDtypeStruct((1, o), f32),
        scratch_shapes=[pltpu.VMEM((n, k), bf16),
                        pltpu.VMEM((n, d), bf16),
                        pltpu.VMEM((n, d), f32),
                        pltpu.VMEM((k, d), f32),
                        pltpu.VMEM((k, 1), f32)],
    )(hbt, x, dv, de, Wp.T, bp[None, :], W1.T,
      b1[None, :], Wr.T, br[None, :])

    return out[0]
